# Initial kernel scaffold; baseline (speedup 1.0000x reference)
import jax
import jax.numpy as jnp
from jax.experimental import pallas as pl


def kernel(user, venue, user_embeddings, venue_embeddings):
    raise NotImplementedError("write your pallas kernel here")


# speedup vs baseline: 82.7242x; 82.7242x over previous
import functools

import jax
import jax.numpy as jnp
from jax import lax
from jax.experimental import pallas as pl
from jax.experimental.pallas import tpu as pltpu
from jax.experimental.pallas import tpu_sc as plsc

B = 1024
HIST = 20
V = 100000
D = 128
R = B * 3
W = 8
VB = 2048
VPAD = 100352
NVB = VPAD // VB
CPB = VB // W
NCHUNK = VPAD // W
K = 20
CAND = K * W


def _sc_gather(table, idx):
    info = plsc.get_sparse_core_info()
    nc, ns = info.num_cores, info.num_subcores
    nw = nc * ns
    n = idx.shape[0]
    b_per_w = n // nw
    chunk = min(b_per_w, 512)
    nch = b_per_w // chunk
    mesh = plsc.VectorSubcoreMesh(core_axis_name="c", subcore_axis_name="s")

    @functools.partial(
        pl.kernel,
        mesh=mesh,
        out_type=jax.ShapeDtypeStruct((n, D), jnp.float32),
        scratch_types=[
            pltpu.VMEM((chunk,), jnp.int32),
            pltpu.VMEM((chunk, D), jnp.float32),
            pltpu.SemaphoreType.DMA,
        ],
    )
    def gather_kernel(table_hbm, idx_hbm, out_hbm, idx_v, rows_v, sem):
        wid = lax.axis_index("s") * nc + lax.axis_index("c")
        base = wid * b_per_w

        def body(i, carry):
            off = base + i * chunk
            pltpu.sync_copy(idx_hbm.at[pl.ds(off, chunk)], idx_v)
            pltpu.async_copy(table_hbm.at[idx_v], rows_v, sem).wait()
            pltpu.sync_copy(rows_v, out_hbm.at[pl.ds(off, chunk)])
            return carry

        lax.fori_loop(0, nch, body, 0)

    return gather_kernel(table, idx)


def _score_chunkmax(ue, vemb_pad):
    RB = 1024

    def body(v_ref, u_ref, m_ref):
        v = pl.program_id(1)
        s = lax.dot_general(
            v_ref[...], u_ref[...],
            (((1,), (1,)), ((), ())),
            preferred_element_type=jnp.float32,
        )
        vid = v * VB + lax.broadcasted_iota(jnp.int32, (VB, 1), 0)
        s = jnp.where(vid < V, s, -jnp.inf)
        m_ref[...] = jnp.max(s.reshape(CPB, W, RB), axis=1)

    return pl.pallas_call(
        body,
        grid=(R // RB, NVB),
        in_specs=[
            pl.BlockSpec((VB, D), lambda r, v: (v, 0)),
            pl.BlockSpec((RB, D), lambda r, v: (r, 0)),
        ],
        out_specs=pl.BlockSpec((CPB, RB), lambda r, v: (v, r)),
        out_shape=jax.ShapeDtypeStruct((NCHUNK, R), jnp.float32),
        compiler_params=pltpu.CompilerParams(
            dimension_semantics=("arbitrary", "arbitrary"),
        ),
    )(vemb_pad, ue)


def _top_chunks(mt):
    RB = 256

    def body(m_ref, o_ref, w_ref):
        w_ref[...] = m_ref[...]
        cio = lax.broadcasted_iota(jnp.int32, (NCHUNK, RB), 0)
        for k in range(K):
            wv = w_ref[...]
            m = jnp.max(wv, axis=0, keepdims=True)
            eq = wv == m
            cid = jnp.min(jnp.where(eq, cio, NCHUNK), axis=0,
                          keepdims=True)
            o_ref[k:k + 1, :] = cid
            w_ref[...] = jnp.where(cio == cid, -jnp.inf, wv)

    return pl.pallas_call(
        body,
        grid=(R // RB,),
        in_specs=[pl.BlockSpec((NCHUNK, RB), lambda r: (0, r))],
        out_specs=pl.BlockSpec((32, RB), lambda r: (0, r)),
        out_shape=jax.ShapeDtypeStruct((32, R), jnp.int32),
        scratch_shapes=[pltpu.VMEM((NCHUNK, RB), jnp.float32)],
    )(mt)


def _final_topk(ue, cand_emb, cand_ids, targets):
    RB = 64

    def body(u_ref, ce_ref, ci_ref, t_ref, idx_ref, cnt_ref):
        r = pl.program_id(0)
        st = lax.dot_general(
            ce_ref[...], u_ref[...],
            (((1,), (1,)), ((), ())),
            preferred_element_type=jnp.float32,
        )
        s3 = st.reshape(RB, CAND, RB)
        eye = (lax.broadcasted_iota(jnp.int32, (RB, 1, RB), 0)
               == lax.broadcasted_iota(jnp.int32, (RB, 1, RB), 2))
        s = jnp.sum(jnp.where(eye, s3, 0.0), axis=2)
        ids = ci_ref[...]
        tgt = t_ref[...]

        work = s
        vids = []
        for k in range(K):
            m = jnp.max(work, axis=1, keepdims=True)
            eq = work == m
            vid = jnp.min(jnp.where(eq, ids, jnp.int32(2 ** 30)), axis=1,
                          keepdims=True)
            vids.append(vid)
            work = jnp.where(ids == vid, -jnp.inf, work)

        topm = jnp.concatenate(vids, axis=1)
        idx_ref[...] = topm

        eqm = (topm == tgt).astype(jnp.int32)
        kio = lax.broadcasted_iota(jnp.int32, (RB, K), 1)
        c01 = jnp.sum(eqm * (kio < 1))
        c05 = jnp.sum(eqm * (kio < 5))
        c10 = jnp.sum(eqm * (kio < 10))
        c20 = jnp.sum(eqm)

        @pl.when(r == 0)
        def _():
            cnt_ref[...] = jnp.zeros_like(cnt_ref)

        lane = lax.broadcasted_iota(jnp.int32, (1, 128), 1)
        add = (jnp.where(lane == 0, c01, 0) + jnp.where(lane == 1, c05, 0)
               + jnp.where(lane == 2, c10, 0) + jnp.where(lane == 3, c20, 0))
        cnt_ref[0:1, :] = cnt_ref[0:1, :] + add

    return pl.pallas_call(
        body,
        grid=(R // RB,),
        in_specs=[
            pl.BlockSpec((RB, D), lambda r: (r, 0)),
            pl.BlockSpec((RB * CAND, D), lambda r: (r, 0)),
            pl.BlockSpec((RB, CAND), lambda r: (r, 0)),
            pl.BlockSpec((RB, 1), lambda r: (r, 0)),
        ],
        out_specs=[
            pl.BlockSpec((RB, K), lambda r: (r, 0)),
            pl.BlockSpec((8, 128), lambda r: (0, 0)),
        ],
        out_shape=[
            jax.ShapeDtypeStruct((R, K), jnp.int32),
            jax.ShapeDtypeStruct((8, 128), jnp.int32),
        ],
        compiler_params=pltpu.CompilerParams(
            dimension_semantics=("arbitrary",),
        ),
    )(ue, cand_emb, cand_ids, targets)


def kernel(user, venue, user_embeddings, venue_embeddings):
    uidx = user.reshape(R).astype(jnp.int32)
    ue = _sc_gather(user_embeddings, uidx)

    vpad = jnp.pad(venue_embeddings, ((0, VPAD - V), (0, 0)))
    mt = _score_chunkmax(ue, vpad)

    cid32 = _top_chunks(mt)
    cids = cid32[:K].T
    vids = (cids[:, :, None] * W
            + jnp.arange(W, dtype=jnp.int32)).reshape(R, CAND)

    cand_emb = _sc_gather(venue_embeddings, vids.reshape(-1))

    targets = venue[:, -3:].reshape(R, 1).astype(jnp.int32)
    top_idx, cnts = _final_topk(ue, cand_emb, vids, targets)

    top_idx = top_idx.reshape(B, 3, K)
    c = cnts[0]
    return (top_idx, c[0], c[1], c[2], c[3], jnp.int32(R))


# speedup vs baseline: 108.8851x; 1.3162x over previous
import functools

import jax
import jax.numpy as jnp
from jax import lax
from jax.experimental import pallas as pl
from jax.experimental.pallas import tpu as pltpu
from jax.experimental.pallas import tpu_sc as plsc

B = 1024
HIST = 20
V = 100000
D = 128
R = B * 3
W = 8
VB = 2048
VPAD = 100352
NVB = VPAD // VB
CPB = VB // W
NCHUNK = VPAD // W
SUP = 128
NSUP = NCHUNK // SUP
NSUP_P = 104
SPB = CPB // SUP
K = 20
CAND2 = K * SUP
CAND = K * W


def _divisor_chunk(n, cap=512):
    c = min(n, cap)
    while n % c or c % 8:
        c -= 8
    return c


def _sc_gather(table, idx, d):
    info = plsc.get_sparse_core_info()
    nc, ns = info.num_cores, info.num_subcores
    nw = nc * ns
    n = idx.shape[0]
    b_per_w = n // nw
    chunk = _divisor_chunk(b_per_w)
    nch = b_per_w // chunk
    mesh = plsc.VectorSubcoreMesh(core_axis_name="c", subcore_axis_name="s")

    @functools.partial(
        pl.kernel,
        mesh=mesh,
        out_type=jax.ShapeDtypeStruct((n, d), jnp.float32),
        scratch_types=[
            pltpu.VMEM((chunk,), jnp.int32),
            pltpu.VMEM((chunk, d), jnp.float32),
            pltpu.SemaphoreType.DMA,
        ],
    )
    def gather_kernel(table_hbm, idx_hbm, out_hbm, idx_v, rows_v, sem):
        wid = lax.axis_index("s") * nc + lax.axis_index("c")
        base = wid * b_per_w

        def body(i, carry):
            off = base + i * chunk
            pltpu.sync_copy(idx_hbm.at[pl.ds(off, chunk)], idx_v)
            pltpu.async_copy(table_hbm.at[idx_v], rows_v, sem).wait()
            pltpu.sync_copy(rows_v, out_hbm.at[pl.ds(off, chunk)])
            return carry

        lax.fori_loop(0, nch, body, 0)

    return gather_kernel(table, idx)


def _score_chunkmax(ue, vemb_pad):
    RB = 1024

    def body(v_ref, u_ref, m_ref, m2_ref):
        v = pl.program_id(1)
        s = lax.dot_general(
            v_ref[...], u_ref[...],
            (((1,), (1,)), ((), ())),
            preferred_element_type=jnp.float32,
        )
        vid = v * VB + lax.broadcasted_iota(jnp.int32, (VB, 1), 0)
        s = jnp.where(vid < V, s, -jnp.inf)
        cm = jnp.max(s.reshape(CPB, W, RB), axis=1)
        m_ref[...] = cm.T
        m2_ref[...] = jnp.max(cm.reshape(SPB, SUP, RB),
                              axis=1)[None]

    return pl.pallas_call(
        body,
        grid=(R // RB, NVB),
        in_specs=[
            pl.BlockSpec((VB, D), lambda r, v: (v, 0)),
            pl.BlockSpec((RB, D), lambda r, v: (r, 0)),
        ],
        out_specs=[
            pl.BlockSpec((RB, CPB), lambda r, v: (r, v)),
            pl.BlockSpec((1, SPB, RB), lambda r, v: (v, 0, r)),
        ],
        out_shape=[
            jax.ShapeDtypeStruct((R, NCHUNK), jnp.float32),
            jax.ShapeDtypeStruct((NVB, SPB, R), jnp.float32),
        ],
        compiler_params=pltpu.CompilerParams(
            dimension_semantics=("arbitrary", "arbitrary"),
        ),
    )(vemb_pad, ue)


def _top_superchunks(m2):
    RB = 512

    def body(m_ref, o_ref, w_ref):
        w_ref[...] = m_ref[...]
        sio = lax.broadcasted_iota(jnp.int32, (NSUP, RB), 0)
        for k in range(K):
            wv = w_ref[...]
            m = jnp.max(wv, axis=0, keepdims=True)
            eq = wv == m
            sid = jnp.min(jnp.where(eq, sio, NSUP), axis=0,
                          keepdims=True)
            o_ref[k:k + 1, :] = sid
            w_ref[...] = jnp.where(sio == sid, -jnp.inf, wv)

    return pl.pallas_call(
        body,
        grid=(R // RB,),
        in_specs=[pl.BlockSpec((NSUP, RB), lambda r: (0, r))],
        out_specs=pl.BlockSpec((32, RB), lambda r: (0, r)),
        out_shape=jax.ShapeDtypeStruct((32, R), jnp.int32),
        scratch_shapes=[pltpu.VMEM((NSUP, RB), jnp.float32)],
    )(m2)


def _top_chunks(g, cids2):
    RB = 512

    def body(g_ref, c_ref, o_ref):
        work = g_ref[...]
        ids = c_ref[...]
        outs = []
        for k in range(K):
            m = jnp.max(work, axis=1, keepdims=True)
            eq = work == m
            cid = jnp.min(jnp.where(eq, ids, jnp.int32(2 ** 30)), axis=1,
                          keepdims=True)
            outs.append(cid)
            work = jnp.where(ids == cid, -jnp.inf, work)
        o_ref[...] = jnp.concatenate(outs, axis=1)

    return pl.pallas_call(
        body,
        grid=(R // RB,),
        in_specs=[
            pl.BlockSpec((RB, CAND2), lambda r: (r, 0)),
            pl.BlockSpec((RB, CAND2), lambda r: (r, 0)),
        ],
        out_specs=pl.BlockSpec((RB, K), lambda r: (r, 0)),
        out_shape=jax.ShapeDtypeStruct((R, K), jnp.int32),
    )(g, cids2)


def _final_topk(ue, cand_emb, cand_ids, targets):
    RB = 64

    def body(u_ref, ce_ref, ci_ref, t_ref, idx_ref, cnt_ref):
        r = pl.program_id(0)
        st = lax.dot_general(
            ce_ref[...], u_ref[...],
            (((1,), (1,)), ((), ())),
            preferred_element_type=jnp.float32,
        )
        s3 = st.reshape(RB, CAND, RB)
        eye = (lax.broadcasted_iota(jnp.int32, (RB, 1, RB), 0)
               == lax.broadcasted_iota(jnp.int32, (RB, 1, RB), 2))
        s = jnp.sum(jnp.where(eye, s3, 0.0), axis=2)
        ids = ci_ref[...]
        tgt = t_ref[...]

        work = s
        vids = []
        for k in range(K):
            m = jnp.max(work, axis=1, keepdims=True)
            eq = work == m
            vid = jnp.min(jnp.where(eq, ids, jnp.int32(2 ** 30)), axis=1,
                          keepdims=True)
            vids.append(vid)
            work = jnp.where(ids == vid, -jnp.inf, work)

        topm = jnp.concatenate(vids, axis=1)
        idx_ref[...] = topm

        eqm = (topm == tgt).astype(jnp.int32)
        kio = lax.broadcasted_iota(jnp.int32, (RB, K), 1)
        c01 = jnp.sum(eqm * (kio < 1))
        c05 = jnp.sum(eqm * (kio < 5))
        c10 = jnp.sum(eqm * (kio < 10))
        c20 = jnp.sum(eqm)

        @pl.when(r == 0)
        def _():
            cnt_ref[...] = jnp.zeros_like(cnt_ref)

        lane = lax.broadcasted_iota(jnp.int32, (1, 128), 1)
        add = (jnp.where(lane == 0, c01, 0) + jnp.where(lane == 1, c05, 0)
               + jnp.where(lane == 2, c10, 0) + jnp.where(lane == 3, c20, 0))
        cnt_ref[0:1, :] = cnt_ref[0:1, :] + add

    return pl.pallas_call(
        body,
        grid=(R // RB,),
        in_specs=[
            pl.BlockSpec((RB, D), lambda r: (r, 0)),
            pl.BlockSpec((RB * CAND, D), lambda r: (r, 0)),
            pl.BlockSpec((RB, CAND), lambda r: (r, 0)),
            pl.BlockSpec((RB, 1), lambda r: (r, 0)),
        ],
        out_specs=[
            pl.BlockSpec((RB, K), lambda r: (r, 0)),
            pl.BlockSpec((8, 128), lambda r: (0, 0)),
        ],
        out_shape=[
            jax.ShapeDtypeStruct((R, K), jnp.int32),
            jax.ShapeDtypeStruct((8, 128), jnp.int32),
        ],
        compiler_params=pltpu.CompilerParams(
            dimension_semantics=("arbitrary",),
        ),
    )(ue, cand_emb, cand_ids, targets)


def kernel(user, venue, user_embeddings, venue_embeddings):
    uidx = user.reshape(R).astype(jnp.int32)
    ue = _sc_gather(user_embeddings, uidx, D)

    vpad = jnp.pad(venue_embeddings, ((0, VPAD - V), (0, 0)))
    m_rows, m2 = _score_chunkmax(ue, vpad)

    sup32 = _top_superchunks(m2.reshape(NSUP, R))
    sups = sup32[:K].T
    gidx = (jnp.arange(R, dtype=jnp.int32)[:, None] * NSUP
            + sups).reshape(-1)
    g = _sc_gather(m_rows.reshape(R * NSUP, SUP), gidx,
                   SUP).reshape(R, CAND2)
    cids2 = (sups[:, :, None] * SUP
             + jnp.arange(SUP, dtype=jnp.int32)).reshape(R, CAND2)

    cids = _top_chunks(g, cids2)
    vids = (cids[:, :, None] * W
            + jnp.arange(W, dtype=jnp.int32)).reshape(R, CAND)

    cand_emb = _sc_gather(venue_embeddings, vids.reshape(-1), D)

    targets = venue[:, -3:].reshape(R, 1).astype(jnp.int32)
    top_idx, cnts = _final_topk(ue, cand_emb, vids, targets)

    top_idx = top_idx.reshape(B, 3, K)
    c = cnts[0]
    return (top_idx, c[0], c[1], c[2], c[3], jnp.int32(R))


# speedup vs baseline: 117.9239x; 1.0830x over previous
import functools

import jax
import jax.numpy as jnp
from jax import lax
from jax.experimental import pallas as pl
from jax.experimental.pallas import tpu as pltpu
from jax.experimental.pallas import tpu_sc as plsc

B = 1024
HIST = 20
V = 100000
D = 128
R = B * 3
W = 8
VB = 2048
VPAD = 100352
NVB = VPAD // VB
CPB = VB // W
NCHUNK = VPAD // W
SUP = 128
NSUP = NCHUNK // SUP
NSUP_P = 104
SPB = CPB // SUP
K = 20
CAND2 = K * SUP
CAND = K * W


def _divisor_chunk(n, cap=512):
    c = min(n, cap)
    while n % c or c % 8:
        c -= 8
    return c


def _sc_gather(table, idx, d):
    info = plsc.get_sparse_core_info()
    nc, ns = info.num_cores, info.num_subcores
    nw = nc * ns
    n = idx.shape[0]
    b_per_w = n // nw
    chunk = _divisor_chunk(b_per_w)
    nch = b_per_w // chunk
    mesh = plsc.VectorSubcoreMesh(core_axis_name="c", subcore_axis_name="s")

    @functools.partial(
        pl.kernel,
        mesh=mesh,
        out_type=jax.ShapeDtypeStruct((n, d), jnp.float32),
        scratch_types=[
            pltpu.VMEM((chunk,), jnp.int32),
            pltpu.VMEM((chunk, d), jnp.float32),
            pltpu.SemaphoreType.DMA,
        ],
    )
    def gather_kernel(table_hbm, idx_hbm, out_hbm, idx_v, rows_v, sem):
        wid = lax.axis_index("s") * nc + lax.axis_index("c")
        base = wid * b_per_w

        def body(i, carry):
            off = base + i * chunk
            pltpu.sync_copy(idx_hbm.at[pl.ds(off, chunk)], idx_v)
            pltpu.async_copy(table_hbm.at[idx_v], rows_v, sem).wait()
            pltpu.sync_copy(rows_v, out_hbm.at[pl.ds(off, chunk)])
            return carry

        lax.fori_loop(0, nch, body, 0)

    return gather_kernel(table, idx)


def _score_chunkmax(ue, vemb_pad, rows, RB):
    def body(v_ref, u_ref, m_ref, m2_ref):
        v = pl.program_id(1)
        s = lax.dot_general(
            v_ref[...], u_ref[...],
            (((1,), (1,)), ((), ())),
            preferred_element_type=jnp.float32,
        )
        vid = v * VB + lax.broadcasted_iota(jnp.int32, (VB, 1), 0)
        s = jnp.where(vid < V, s, -jnp.inf)
        cm = jnp.max(s.reshape(CPB, W, RB), axis=1)
        m_ref[...] = cm.T
        m2_ref[...] = jnp.max(cm.reshape(SPB, SUP, RB),
                              axis=1)[None]

    return pl.pallas_call(
        body,
        grid=(rows // RB, NVB),
        in_specs=[
            pl.BlockSpec((VB, D), lambda r, v: (v, 0)),
            pl.BlockSpec((RB, D), lambda r, v: (r, 0)),
        ],
        out_specs=[
            pl.BlockSpec((RB, CPB), lambda r, v: (r, v)),
            pl.BlockSpec((1, SPB, RB), lambda r, v: (v, 0, r)),
        ],
        out_shape=[
            jax.ShapeDtypeStruct((rows, NCHUNK), jnp.float32),
            jax.ShapeDtypeStruct((NVB, SPB, rows), jnp.float32),
        ],
        compiler_params=pltpu.CompilerParams(
            dimension_semantics=("arbitrary", "arbitrary"),
        ),
    )(vemb_pad, ue)


def _top_superchunks(m2, rows):
    RB = 512

    def body(m_ref, o_ref, w_ref):
        w_ref[...] = m_ref[...]
        sio = lax.broadcasted_iota(jnp.int32, (NSUP, RB), 0)
        for k in range(K):
            wv = w_ref[...]
            m = jnp.max(wv, axis=0, keepdims=True)
            eq = wv == m
            sid = jnp.min(jnp.where(eq, sio, NSUP), axis=0,
                          keepdims=True)
            o_ref[k:k + 1, :] = sid
            w_ref[...] = jnp.where(sio == sid, -jnp.inf, wv)

    return pl.pallas_call(
        body,
        grid=(rows // RB,),
        in_specs=[pl.BlockSpec((NSUP, RB), lambda r: (0, r))],
        out_specs=pl.BlockSpec((32, RB), lambda r: (0, r)),
        out_shape=jax.ShapeDtypeStruct((32, rows), jnp.int32),
        scratch_shapes=[pltpu.VMEM((NSUP, RB), jnp.float32)],
    )(m2)


def _top_chunks(g, cids2, rows):
    RB = 512

    def body(g_ref, c_ref, o_ref):
        work = g_ref[...]
        ids = c_ref[...]
        outs = []
        for k in range(K):
            m = jnp.max(work, axis=1, keepdims=True)
            eq = work == m
            cid = jnp.min(jnp.where(eq, ids, jnp.int32(2 ** 30)), axis=1,
                          keepdims=True)
            outs.append(cid)
            work = jnp.where(ids == cid, -jnp.inf, work)
        o_ref[...] = jnp.concatenate(outs, axis=1)

    return pl.pallas_call(
        body,
        grid=(rows // RB,),
        in_specs=[
            pl.BlockSpec((RB, CAND2), lambda r: (r, 0)),
            pl.BlockSpec((RB, CAND2), lambda r: (r, 0)),
        ],
        out_specs=pl.BlockSpec((RB, K), lambda r: (r, 0)),
        out_shape=jax.ShapeDtypeStruct((rows, K), jnp.int32),
    )(g, cids2)


def _final_topk(ue, cand_emb, cand_ids, targets, rows):
    RB = 64

    def body(u_ref, ce_ref, ci_ref, t_ref, idx_ref, cnt_ref):
        r = pl.program_id(0)
        st = lax.dot_general(
            ce_ref[...], u_ref[...],
            (((1,), (1,)), ((), ())),
            preferred_element_type=jnp.float32,
        )
        s3 = st.reshape(RB, CAND, RB)
        eye = (lax.broadcasted_iota(jnp.int32, (RB, 1, RB), 0)
               == lax.broadcasted_iota(jnp.int32, (RB, 1, RB), 2))
        s = jnp.sum(jnp.where(eye, s3, 0.0), axis=2)
        ids = ci_ref[...]
        tgt = t_ref[...]

        work = s
        vids = []
        for k in range(K):
            m = jnp.max(work, axis=1, keepdims=True)
            eq = work == m
            vid = jnp.min(jnp.where(eq, ids, jnp.int32(2 ** 30)), axis=1,
                          keepdims=True)
            vids.append(vid)
            work = jnp.where(ids == vid, -jnp.inf, work)

        topm = jnp.concatenate(vids, axis=1)
        idx_ref[...] = topm

        eqm = (topm == tgt).astype(jnp.int32)
        kio = lax.broadcasted_iota(jnp.int32, (RB, K), 1)
        c01 = jnp.sum(eqm * (kio < 1))
        c05 = jnp.sum(eqm * (kio < 5))
        c10 = jnp.sum(eqm * (kio < 10))
        c20 = jnp.sum(eqm)

        @pl.when(r == 0)
        def _():
            cnt_ref[...] = jnp.zeros_like(cnt_ref)

        lane = lax.broadcasted_iota(jnp.int32, (1, 128), 1)
        add = (jnp.where(lane == 0, c01, 0) + jnp.where(lane == 1, c05, 0)
               + jnp.where(lane == 2, c10, 0) + jnp.where(lane == 3, c20, 0))
        cnt_ref[0:1, :] = cnt_ref[0:1, :] + add

    return pl.pallas_call(
        body,
        grid=(rows // RB,),
        in_specs=[
            pl.BlockSpec((RB, D), lambda r: (r, 0)),
            pl.BlockSpec((RB * CAND, D), lambda r: (r, 0)),
            pl.BlockSpec((RB, CAND), lambda r: (r, 0)),
            pl.BlockSpec((RB, 1), lambda r: (r, 0)),
        ],
        out_specs=[
            pl.BlockSpec((RB, K), lambda r: (r, 0)),
            pl.BlockSpec((8, 128), lambda r: (0, 0)),
        ],
        out_shape=[
            jax.ShapeDtypeStruct((rows, K), jnp.int32),
            jax.ShapeDtypeStruct((8, 128), jnp.int32),
        ],
        compiler_params=pltpu.CompilerParams(
            dimension_semantics=("arbitrary",),
        ),
    )(ue, cand_emb, cand_ids, targets)


def _stripe(ue_s, vpad, venue_embeddings, targets_s, rows):
    m_rows, m2 = _score_chunkmax(ue_s, vpad, rows, 768)

    sup32 = _top_superchunks(m2.reshape(NSUP, rows), rows)
    sups = sup32[:K].T
    gidx = (jnp.arange(rows, dtype=jnp.int32)[:, None] * NSUP
            + sups).reshape(-1)
    g = _sc_gather(m_rows.reshape(rows * NSUP, SUP), gidx,
                   SUP).reshape(rows, CAND2)
    cids2 = (sups[:, :, None] * SUP
             + jnp.arange(SUP, dtype=jnp.int32)).reshape(rows, CAND2)

    cids = _top_chunks(g, cids2, rows)
    vids = (cids[:, :, None] * W
            + jnp.arange(W, dtype=jnp.int32)).reshape(rows, CAND)

    cand_emb = _sc_gather(venue_embeddings, vids.reshape(-1), D)
    return _final_topk(ue_s, cand_emb, vids, targets_s, rows)


def kernel(user, venue, user_embeddings, venue_embeddings):
    uidx = user.reshape(R).astype(jnp.int32)
    ue = _sc_gather(user_embeddings, uidx, D)

    vpad = jnp.pad(venue_embeddings, ((0, VPAD - V), (0, 0)))
    targets = venue[:, -3:].reshape(R, 1).astype(jnp.int32)

    NS = 2
    RS = R // NS
    parts = [
        _stripe(ue[i * RS:(i + 1) * RS], vpad, venue_embeddings,
                targets[i * RS:(i + 1) * RS], RS)
        for i in range(NS)
    ]

    top_idx = jnp.concatenate([p[0] for p in parts]).reshape(B, 3, K)
    c = sum(p[1] for p in parts)[0]
    return (top_idx, c[0], c[1], c[2], c[3], jnp.int32(R))
